# baseline (speedup 1.0000x reference)
import functools

import jax
import jax.numpy as jnp
from jax.experimental import pallas as pl
from jax.experimental.pallas import tpu as pltpu

_NUM_EXPERTS = 16
_TOP_K = 2
_LANES = 128
_NEG = -1e30
_NSPLIT = 2


def _top2(x, w, b, lane):
    logits = jax.lax.dot_general(
        x, w, (((1,), (0,)), ((), ())),
        preferred_element_type=jnp.float32) + b
    logits = jnp.where(lane < _NUM_EXPERTS, logits, _NEG)
    m1 = jnp.max(logits, axis=1, keepdims=True)
    i1 = jnp.min(jnp.where(logits == m1, lane, _LANES), axis=1, keepdims=True)
    rest = jnp.where(lane == i1, _NEG, logits)
    m2 = jnp.max(rest, axis=1, keepdims=True)
    i2 = jnp.min(jnp.where(rest == m2, lane, _LANES), axis=1, keepdims=True)
    w1 = 1.0 / (1.0 + jnp.exp(m2 - m1))
    cnt = (jnp.sum(jnp.where(lane == i1, 1.0, 0.0), axis=0, keepdims=True) +
           jnp.sum(jnp.where(lane == i2, 1.0, 0.0), axis=0, keepdims=True))
    return (jnp.concatenate([i1, i2], axis=1),
            jnp.concatenate([w1, 1.0 - w1], axis=1), cnt)


def _router_body(num_tiles, x0_ref, x1_ref, w_ref, b_ref, idx0_ref, wgt0_ref,
                 idx1_ref, wgt1_ref, loss_ref, cnt_ref):
    step = pl.program_id(0)
    tile = x0_ref.shape[0]
    lane = jax.lax.broadcasted_iota(jnp.int32, (tile, _LANES), 1)
    w = w_ref[...]
    b = b_ref[...]

    idx0, wgt0, c0 = _top2(x0_ref[...], w, b, lane)
    idx0_ref[...] = idx0
    wgt0_ref[...] = wgt0
    idx1, wgt1, c1 = _top2(x1_ref[...], w, b, lane)
    idx1_ref[...] = idx1
    wgt1_ref[...] = wgt1
    c = c0 + c1

    @pl.when(step == 0)
    def _():
        cnt_ref[...] = c

    @pl.when(step > 0)
    def _():
        cnt_ref[...] = cnt_ref[...] + c

    @pl.when(step == num_tiles - 1)
    def _():
        cnts = cnt_ref[...]
        mean = jnp.sum(cnts) / _NUM_EXPERTS
        emask = lane[0:1, :] < _NUM_EXPERTS
        var = jnp.sum(jnp.where(emask, (cnts - mean) ** 2, 0.0)) / (
            _NUM_EXPERTS - 1)
        loss_ref[...] = jnp.reshape(jnp.sqrt(var) / (mean + 1e-10) * 0.01,
                                    (1, 1))


@jax.jit
def kernel(x, W, b):
    B, S, D = x.shape
    T = B * S
    half = T // _NSPLIT
    xf = x.reshape(T, D)

    Wp = jnp.zeros((D, _LANES), jnp.float32).at[:, :_NUM_EXPERTS].set(W)
    bp = jnp.zeros((1, _LANES), jnp.float32).at[0, :_NUM_EXPERTS].set(b)

    tile = 1024
    num_tiles = half // tile

    tok_spec = pl.BlockSpec((tile, D), lambda i: (i, 0))
    out_spec = pl.BlockSpec((tile, _TOP_K), lambda i: (i, 0))
    out_tok = [
        jax.ShapeDtypeStruct((half, _TOP_K), jnp.int32),
        jax.ShapeDtypeStruct((half, _TOP_K), jnp.float32),
    ]

    idx0, wgt0, idx1, wgt1, loss = pl.pallas_call(
        functools.partial(_router_body, num_tiles),
        grid=(num_tiles,),
        in_specs=[
            tok_spec,
            tok_spec,
            pl.BlockSpec((D, _LANES), lambda i: (0, 0)),
            pl.BlockSpec((1, _LANES), lambda i: (0, 0)),
        ],
        out_specs=[out_spec, out_spec, out_spec, out_spec,
                   pl.BlockSpec((1, 1), lambda i: (0, 0))],
        out_shape=out_tok + out_tok + [
            jax.ShapeDtypeStruct((1, 1), jnp.float32)],
        scratch_shapes=[pltpu.VMEM((1, _LANES), jnp.float32)],
    )(xf[:half], xf[half:], Wp, bp)

    idx = jnp.concatenate([idx0, idx1], axis=0).reshape(B, S, _TOP_K)
    wgt = jnp.concatenate([wgt0, wgt1], axis=0).reshape(B, S, _TOP_K)
    return (idx, wgt, loss[0, 0])


# speedup vs baseline: 2.2372x; 2.2372x over previous
import functools

import jax
import jax.numpy as jnp
from jax.experimental import pallas as pl
from jax.experimental.pallas import tpu as pltpu

_NUM_EXPERTS = 16
_TOP_K = 2
_LANES = 128
_NEG = -1e30
_NSPLIT = 2


def _top2(x, w, b, lane):
    logits = jax.lax.dot_general(
        x, w, (((1,), (0,)), ((), ())),
        preferred_element_type=jnp.float32) + b
    logits = jnp.where(lane < _NUM_EXPERTS, logits, _NEG)
    m1 = jnp.max(logits, axis=1, keepdims=True)
    i1 = jnp.min(jnp.where(logits == m1, lane, _LANES), axis=1, keepdims=True)
    rest = jnp.where(lane == i1, _NEG, logits)
    m2 = jnp.max(rest, axis=1, keepdims=True)
    i2 = jnp.min(jnp.where(rest == m2, lane, _LANES), axis=1, keepdims=True)
    w1 = 1.0 / (1.0 + jnp.exp(m2 - m1))
    cnt = (jnp.sum(jnp.where(lane == i1, 1.0, 0.0), axis=0, keepdims=True) +
           jnp.sum(jnp.where(lane == i2, 1.0, 0.0), axis=0, keepdims=True))
    return (jnp.concatenate([i1, i2], axis=1),
            jnp.concatenate([w1, 1.0 - w1], axis=1), cnt)


def _router_body(num_tiles, x0_ref, x1_ref, w_ref, b_ref, idx0_ref, wgt0_ref,
                 idx1_ref, wgt1_ref, loss_ref, cnt_ref):
    step = pl.program_id(0)
    tile = x0_ref.shape[1]
    lane = jax.lax.broadcasted_iota(jnp.int32, (tile, _LANES), 1)
    w = w_ref[...]
    b = b_ref[...]

    idx0, wgt0, c0 = _top2(x0_ref[0], w, b, lane)
    idx0_ref[...] = idx0
    wgt0_ref[...] = wgt0
    idx1, wgt1, c1 = _top2(x1_ref[0], w, b, lane)
    idx1_ref[...] = idx1
    wgt1_ref[...] = wgt1
    c = c0 + c1

    @pl.when(step == 0)
    def _():
        cnt_ref[...] = c

    @pl.when(step > 0)
    def _():
        cnt_ref[...] = cnt_ref[...] + c

    @pl.when(step == num_tiles - 1)
    def _():
        cnts = cnt_ref[...]
        mean = jnp.sum(cnts) / _NUM_EXPERTS
        emask = lane[0:1, :] < _NUM_EXPERTS
        var = jnp.sum(jnp.where(emask, (cnts - mean) ** 2, 0.0)) / (
            _NUM_EXPERTS - 1)
        loss_ref[...] = jnp.reshape(jnp.sqrt(var) / (mean + 1e-10) * 0.01,
                                    (1, 1))


@jax.jit
def kernel(x, W, b):
    B, S, D = x.shape
    T = B * S
    half = T // _NSPLIT
    xf = x.reshape(T, D)

    Wp = jnp.zeros((D, _LANES), jnp.float32).at[:, :_NUM_EXPERTS].set(W)
    bp = jnp.zeros((1, _LANES), jnp.float32).at[0, :_NUM_EXPERTS].set(b)

    x2 = xf.reshape(_NSPLIT, half, D)
    tile = 1024
    num_tiles = half // tile

    tok0_spec = pl.BlockSpec((1, tile, D), lambda i: (0, i, 0))
    tok1_spec = pl.BlockSpec((1, tile, D), lambda i: (1, i, 0))
    out_spec = pl.BlockSpec((tile, _TOP_K), lambda i: (i, 0))
    out_tok = [
        jax.ShapeDtypeStruct((half, _TOP_K), jnp.int32),
        jax.ShapeDtypeStruct((half, _TOP_K), jnp.float32),
    ]

    idx0, wgt0, idx1, wgt1, loss = pl.pallas_call(
        functools.partial(_router_body, num_tiles),
        grid=(num_tiles,),
        in_specs=[
            tok0_spec,
            tok1_spec,
            pl.BlockSpec((D, _LANES), lambda i: (0, 0)),
            pl.BlockSpec((1, _LANES), lambda i: (0, 0)),
        ],
        out_specs=[out_spec, out_spec, out_spec, out_spec,
                   pl.BlockSpec((1, 1), lambda i: (0, 0))],
        out_shape=out_tok + out_tok + [
            jax.ShapeDtypeStruct((1, 1), jnp.float32)],
        scratch_shapes=[pltpu.VMEM((1, _LANES), jnp.float32)],
    )(x2, x2, Wp, bp)

    idx = jnp.concatenate([idx0, idx1], axis=0).reshape(B, S, _TOP_K)
    wgt = jnp.concatenate([wgt0, wgt1], axis=0).reshape(B, S, _TOP_K)
    return (idx, wgt, loss[0, 0])
